# baseline (speedup 1.0000x reference)
import functools

import jax
import jax.numpy as jnp
import numpy as np
from jax import lax
from jax.experimental import pallas as pl
from jax.experimental.pallas import tpu as pltpu
from jax.experimental.pallas import tpu_sc as plsc

N = 10000
RADIUS = 2.1 * 0.05
SIGMA = RADIUS
K_NEIGH = 10
KP = 25
OUT_FEATS = 64
POOL_CELL = 0.1
VOXELS = 1000

NPAD = 10240
BQ = 80
GRID = N // BQ


def _tc_body(q_ref, s_ref, sT_ref, kT_ref, w_ref,
             f_ref, pts_ref, cnt_ref, cacc_ref):
    i = pl.program_id(0)
    q = q_ref[...]

    d2 = None
    for c in range(3):
        dc = q[:, c:c + 1] - s_ref[c:c + 1, :]
        d2 = dc * dc if d2 is None else d2 + dc * dc

    k2 = None
    for c in range(3):
        kc = kT_ref[c:c + 1, :]
        k2 = kc * kc if k2 is None else k2 + kc * kc
    infl0 = jnp.maximum(0.0, 1.0 - jnp.sqrt(k2 + 1e-12) / SIGMA)
    infl = jnp.broadcast_to(infl0, (BQ, KP))

    prev = jnp.zeros((BQ, 1), jnp.float32)
    for _ in range(K_NEIGH - 1):
        m = jnp.min(jnp.where(d2 > prev, d2, jnp.float32(3e38)),
                    axis=1, keepdims=True)
        onehot = (d2 == m).astype(jnp.float32)
        prev = m
        nbr = jnp.dot(onehot, sT_ref[...],
                      preferred_element_type=jnp.float32)
        dist2 = None
        for c in range(3):
            dd = (nbr[:, c:c + 1] - q[:, c:c + 1]) - kT_ref[c:c + 1, :]
            dist2 = dd * dd if dist2 is None else dist2 + dd * dd
        dist = jnp.sqrt(dist2 + 1e-12)
        infl = infl + jnp.maximum(0.0, 1.0 - dist / SIGMA)

    f_ref[...] = jnp.dot(infl, w_ref[...],
                         preferred_element_type=jnp.float32)

    gid = jnp.floor(q / POOL_CELL).astype(jnp.int32)
    lin = (gid[:, 0] * 10 + gid[:, 1]) * 10 + gid[:, 2]
    vio = jax.lax.broadcasted_iota(jnp.int32, (BQ, VOXELS), 1)
    oh = (lin[:, None] == vio).astype(jnp.float32)
    rhs = jnp.concatenate([q, jnp.ones((BQ, 1), jnp.float32)], axis=1)
    pacc = jax.lax.dot_general(oh, rhs, (((0,), (0,)), ((), ())),
                               preferred_element_type=jnp.float32)

    @pl.when(i == 0)
    def _init():
        pts_ref[...] = jnp.zeros_like(pts_ref)
        cacc_ref[...] = jnp.zeros_like(cacc_ref)

    pts_ref[...] += pacc[:, :3]
    cacc_ref[...] += pacc[:, 3:4]

    @pl.when(i == pl.num_programs(0) - 1)
    def _final():
        c = cacc_ref[...]
        pts_ref[...] = pts_ref[...] / c
        cnt_ref[...] = c


def _kpconv_features(points1, kern, W1):
    s = points1.T
    s_pad = jnp.pad(s, ((0, 0), (0, NPAD - N)), constant_values=1e3)
    sT_pad = s_pad.T
    kT = kern.T
    w = W1.reshape(KP, OUT_FEATS)

    return pl.pallas_call(
        _tc_body,
        grid=(GRID,),
        in_specs=[
            pl.BlockSpec((BQ, 3), lambda i: (i, 0)),
            pl.BlockSpec((3, NPAD), lambda i: (0, 0)),
            pl.BlockSpec((NPAD, 3), lambda i: (0, 0)),
            pl.BlockSpec((3, KP), lambda i: (0, 0)),
            pl.BlockSpec((KP, OUT_FEATS), lambda i: (0, 0)),
        ],
        out_specs=[
            pl.BlockSpec((BQ, OUT_FEATS), lambda i: (i, 0)),
            pl.BlockSpec((VOXELS, 3), lambda i: (0, 0)),
            pl.BlockSpec((VOXELS, 1), lambda i: (0, 0)),
        ],
        out_shape=[
            jax.ShapeDtypeStruct((N, OUT_FEATS), jnp.float32),
            jax.ShapeDtypeStruct((VOXELS, 3), jnp.float32),
            jax.ShapeDtypeStruct((VOXELS, 1), jnp.float32),
        ],
        scratch_shapes=[pltpu.VMEM((VOXELS, 1), jnp.float32)],
    )(points1, s_pad, sT_pad, kT, w)


NW = 16
CH = NPAD // NW
NCHK = CH // 128
VPAD = 1024
VSLAB = VPAD // NW
FD = 128


def _sc_pool_body(xs_hbm, ys_hbm, zs_hbm, f_hbm, cntb_hbm, z64_hbm,
                  ofeats_hbm, xs_v, ys_v, zs_v, fvs, lins,
                  accf, locf_v, cntb_v, of_v):
    wid = lax.axis_index("s")
    base = wid * CH
    iota16 = lax.broadcasted_iota(jnp.int32, (16,), 0)

    pltpu.sync_copy(z64_hbm, accf.at[pl.ds(wid * VSLAB, VSLAB)])

    pltpu.sync_copy(xs_hbm.at[pl.ds(base, CH)], xs_v)
    pltpu.sync_copy(ys_hbm.at[pl.ds(base, CH)], ys_v)
    pltpu.sync_copy(zs_hbm.at[pl.ds(base, CH)], zs_v)
    for j in range(NCHK):
        pltpu.sync_copy(f_hbm.at[pl.ds(base + j * 128, 128)], fvs[j])
    for g in range(CH // 16):
        vx = xs_v[pl.ds(g * 16, 16)]
        vy = ys_v[pl.ds(g * 16, 16)]
        vz = zs_v[pl.ds(g * 16, 16)]
        gx = (vx / jnp.float32(POOL_CELL)).astype(jnp.int32)
        gy = (vy / jnp.float32(POOL_CELL)).astype(jnp.int32)
        gz = (vz / jnp.float32(POOL_CELL)).astype(jnp.int32)
        lin16 = (gx * 10 + gy) * 10 + gz
        lins[g // 8][pl.ds((g % 8) * 16, 16)] = lin16

    plsc.subcore_barrier()

    for j in range(NCHK):
        pltpu.sync_copy(fvs[j], accf.at[lins[j]], add=True)

    plsc.subcore_barrier()

    vb = wid * VSLAB
    pltpu.sync_copy(accf.at[pl.ds(vb, VSLAB)], locf_v)
    pltpu.sync_copy(cntb_hbm.at[pl.ds(vb, VSLAB)], cntb_v)
    for v in range(VSLAB):
        cb = cntb_v[v, :]
        for cg in range(FD // 16):
            of_v[v, pl.ds(cg * 16, 16)] = (
                locf_v[v, pl.ds(cg * 16, 16)] / cb)
    pltpu.sync_copy(of_v, ofeats_hbm.at[pl.ds(vb, VSLAB)])


def _feat_pool_sc(xs, ys, zs, feats_pad, cnt_bcast):
    mesh = plsc.VectorSubcoreMesh(core_axis_name="c", subcore_axis_name="s",
                                  num_cores=1, num_subcores=NW)
    z64 = jnp.zeros((VSLAB, FD), jnp.float32)

    def body(xs_hbm, ys_hbm, zs_hbm, f_hbm, cntb_hbm, z64_hbm, ofeats_hbm,
             xs_v, ys_v, zs_v,
             f_a, f_b, f_c, f_d, f_e,
             lin_a, lin_b, lin_c, lin_d, lin_e,
             accf, locf_v, cntb_v, of_v):
        _sc_pool_body(xs_hbm, ys_hbm, zs_hbm, f_hbm, cntb_hbm, z64_hbm,
                      ofeats_hbm, xs_v, ys_v, zs_v,
                      [f_a, f_b, f_c, f_d, f_e],
                      [lin_a, lin_b, lin_c, lin_d, lin_e],
                      accf, locf_v, cntb_v, of_v)

    ofeats = pl.kernel(
        body,
        out_type=jax.ShapeDtypeStruct((VPAD, FD), jnp.float32),
        mesh=mesh,
        scratch_types=(
            [pltpu.VMEM((CH,), jnp.float32) for _ in range(3)]
            + [pltpu.VMEM((128, FD), jnp.float32) for _ in range(NCHK)]
            + [pltpu.VMEM((128,), jnp.int32) for _ in range(NCHK)]
            + [pltpu.VMEM_SHARED((VPAD, FD), jnp.float32),
               pltpu.VMEM((VSLAB, FD), jnp.float32),
               pltpu.VMEM((VSLAB, 16), jnp.float32),
               pltpu.VMEM((VSLAB, FD), jnp.float32)]
        ),
    )(xs, ys, zs, feats_pad, cnt_bcast, z64)
    return ofeats[:VOXELS, :OUT_FEATS]


@jax.jit
def kernel(points1, kernel, W1):
    feats2, pts_sub, cnt = _kpconv_features(points1, kernel, W1)
    pT_pad = jnp.pad(points1.T, ((0, 0), (0, NPAD - N)))
    xs, ys, zs = pT_pad[0], pT_pad[1], pT_pad[2]
    f_pad = jnp.pad(feats2, ((0, NPAD - N), (0, FD - OUT_FEATS)))
    cntp = jnp.pad(cnt, ((0, VPAD - VOXELS), (0, 0)))
    cnt_bcast = jnp.broadcast_to(cntp, (VPAD, 16))
    feats_sub = _feat_pool_sc(xs, ys, zs, f_pad, cnt_bcast)
    return pts_sub, feats_sub


# speedup vs baseline: 1.0577x; 1.0577x over previous
import functools

import jax
import jax.numpy as jnp
import numpy as np
from jax import lax
from jax.experimental import pallas as pl
from jax.experimental.pallas import tpu as pltpu
from jax.experimental.pallas import tpu_sc as plsc

N = 10000
RADIUS = 2.1 * 0.05
SIGMA = RADIUS
K_NEIGH = 10
KP = 25
OUT_FEATS = 64
POOL_CELL = 0.1
VOXELS = 1000

NPAD = 10240
BQ = 200
GRID = N // BQ


def _tc_body(q_ref, s_ref, sT_ref, kT_ref, w_ref,
             f_ref, pts_ref, cnt_ref, cacc_ref):
    i = pl.program_id(0)
    q = q_ref[...]

    d2 = None
    for c in range(3):
        dc = q[:, c:c + 1] - s_ref[c:c + 1, :]
        d2 = dc * dc if d2 is None else d2 + dc * dc

    k2 = None
    for c in range(3):
        kc = kT_ref[c:c + 1, :]
        k2 = kc * kc if k2 is None else k2 + kc * kc
    infl0 = jnp.maximum(0.0, 1.0 - jnp.sqrt(k2 + 1e-12) / SIGMA)
    infl = jnp.broadcast_to(infl0, (BQ, KP))

    prev = jnp.zeros((BQ, 1), jnp.float32)
    for _ in range(K_NEIGH - 1):
        m = jnp.min(jnp.where(d2 > prev, d2, jnp.float32(3e38)),
                    axis=1, keepdims=True)
        onehot = (d2 == m).astype(jnp.float32)
        prev = m
        nbr = jnp.dot(onehot, sT_ref[...],
                      preferred_element_type=jnp.float32)
        dist2 = None
        for c in range(3):
            dd = (nbr[:, c:c + 1] - q[:, c:c + 1]) - kT_ref[c:c + 1, :]
            dist2 = dd * dd if dist2 is None else dist2 + dd * dd
        dist = jnp.sqrt(dist2 + 1e-12)
        infl = infl + jnp.maximum(0.0, 1.0 - dist / SIGMA)

    f_ref[...] = jnp.dot(infl, w_ref[...],
                         preferred_element_type=jnp.float32)

    gid = jnp.floor(q / POOL_CELL).astype(jnp.int32)
    lin = (gid[:, 0] * 10 + gid[:, 1]) * 10 + gid[:, 2]
    vio = jax.lax.broadcasted_iota(jnp.int32, (BQ, VOXELS), 1)
    oh = (lin[:, None] == vio).astype(jnp.float32)
    rhs = jnp.concatenate([q, jnp.ones((BQ, 1), jnp.float32)], axis=1)
    pacc = jax.lax.dot_general(oh, rhs, (((0,), (0,)), ((), ())),
                               preferred_element_type=jnp.float32)

    @pl.when(i == 0)
    def _init():
        pts_ref[...] = jnp.zeros_like(pts_ref)
        cacc_ref[...] = jnp.zeros_like(cacc_ref)

    pts_ref[...] += pacc[:, :3]
    cacc_ref[...] += pacc[:, 3:4]

    @pl.when(i == pl.num_programs(0) - 1)
    def _final():
        c = cacc_ref[...]
        pts_ref[...] = pts_ref[...] / c
        cnt_ref[...] = c


def _kpconv_features(points1, kern, W1):
    s = points1.T
    s_pad = jnp.pad(s, ((0, 0), (0, NPAD - N)), constant_values=1e3)
    sT_pad = s_pad.T
    kT = kern.T
    w = W1.reshape(KP, OUT_FEATS)

    return pl.pallas_call(
        _tc_body,
        grid=(GRID,),
        in_specs=[
            pl.BlockSpec((BQ, 3), lambda i: (i, 0)),
            pl.BlockSpec((3, NPAD), lambda i: (0, 0)),
            pl.BlockSpec((NPAD, 3), lambda i: (0, 0)),
            pl.BlockSpec((3, KP), lambda i: (0, 0)),
            pl.BlockSpec((KP, OUT_FEATS), lambda i: (0, 0)),
        ],
        out_specs=[
            pl.BlockSpec((BQ, OUT_FEATS), lambda i: (i, 0)),
            pl.BlockSpec((VOXELS, 3), lambda i: (0, 0)),
            pl.BlockSpec((VOXELS, 1), lambda i: (0, 0)),
        ],
        out_shape=[
            jax.ShapeDtypeStruct((N, OUT_FEATS), jnp.float32),
            jax.ShapeDtypeStruct((VOXELS, 3), jnp.float32),
            jax.ShapeDtypeStruct((VOXELS, 1), jnp.float32),
        ],
        scratch_shapes=[pltpu.VMEM((VOXELS, 1), jnp.float32)],
    )(points1, s_pad, sT_pad, kT, w)


NW = 16
CH = NPAD // NW
NCHK = CH // 128
VPAD = 1024
VSLAB = VPAD // NW
FD = 128


def _sc_pool_body(xs_hbm, ys_hbm, zs_hbm, f_hbm, cntb_hbm, z64_hbm,
                  ofeats_hbm, xs_v, ys_v, zs_v, fvs, lins,
                  accf, locf_v, cntb_v, of_v):
    wid = lax.axis_index("s")
    base = wid * CH
    iota16 = lax.broadcasted_iota(jnp.int32, (16,), 0)

    pltpu.sync_copy(z64_hbm, accf.at[pl.ds(wid * VSLAB, VSLAB)])

    pltpu.sync_copy(xs_hbm.at[pl.ds(base, CH)], xs_v)
    pltpu.sync_copy(ys_hbm.at[pl.ds(base, CH)], ys_v)
    pltpu.sync_copy(zs_hbm.at[pl.ds(base, CH)], zs_v)
    for j in range(NCHK):
        pltpu.sync_copy(f_hbm.at[pl.ds(base + j * 128, 128)], fvs[j])
    for g in range(CH // 16):
        vx = xs_v[pl.ds(g * 16, 16)]
        vy = ys_v[pl.ds(g * 16, 16)]
        vz = zs_v[pl.ds(g * 16, 16)]
        gx = (vx / jnp.float32(POOL_CELL)).astype(jnp.int32)
        gy = (vy / jnp.float32(POOL_CELL)).astype(jnp.int32)
        gz = (vz / jnp.float32(POOL_CELL)).astype(jnp.int32)
        lin16 = (gx * 10 + gy) * 10 + gz
        lins[g // 8][pl.ds((g % 8) * 16, 16)] = lin16

    plsc.subcore_barrier()

    for j in range(NCHK):
        pltpu.sync_copy(fvs[j], accf.at[lins[j]], add=True)

    plsc.subcore_barrier()

    vb = wid * VSLAB
    pltpu.sync_copy(accf.at[pl.ds(vb, VSLAB)], locf_v)
    pltpu.sync_copy(cntb_hbm.at[pl.ds(vb, VSLAB)], cntb_v)
    for v in range(VSLAB):
        cb = cntb_v[v, :]
        for cg in range(FD // 16):
            of_v[v, pl.ds(cg * 16, 16)] = (
                locf_v[v, pl.ds(cg * 16, 16)] / cb)
    pltpu.sync_copy(of_v, ofeats_hbm.at[pl.ds(vb, VSLAB)])


def _feat_pool_sc(xs, ys, zs, feats_pad, cnt_bcast):
    mesh = plsc.VectorSubcoreMesh(core_axis_name="c", subcore_axis_name="s",
                                  num_cores=1, num_subcores=NW)
    z64 = jnp.zeros((VSLAB, FD), jnp.float32)

    def body(xs_hbm, ys_hbm, zs_hbm, f_hbm, cntb_hbm, z64_hbm, ofeats_hbm,
             xs_v, ys_v, zs_v,
             f_a, f_b, f_c, f_d, f_e,
             lin_a, lin_b, lin_c, lin_d, lin_e,
             accf, locf_v, cntb_v, of_v):
        _sc_pool_body(xs_hbm, ys_hbm, zs_hbm, f_hbm, cntb_hbm, z64_hbm,
                      ofeats_hbm, xs_v, ys_v, zs_v,
                      [f_a, f_b, f_c, f_d, f_e],
                      [lin_a, lin_b, lin_c, lin_d, lin_e],
                      accf, locf_v, cntb_v, of_v)

    ofeats = pl.kernel(
        body,
        out_type=jax.ShapeDtypeStruct((VPAD, FD), jnp.float32),
        mesh=mesh,
        scratch_types=(
            [pltpu.VMEM((CH,), jnp.float32) for _ in range(3)]
            + [pltpu.VMEM((128, FD), jnp.float32) for _ in range(NCHK)]
            + [pltpu.VMEM((128,), jnp.int32) for _ in range(NCHK)]
            + [pltpu.VMEM_SHARED((VPAD, FD), jnp.float32),
               pltpu.VMEM((VSLAB, FD), jnp.float32),
               pltpu.VMEM((VSLAB, 16), jnp.float32),
               pltpu.VMEM((VSLAB, FD), jnp.float32)]
        ),
    )(xs, ys, zs, feats_pad, cnt_bcast, z64)
    return ofeats[:VOXELS, :OUT_FEATS]


@jax.jit
def kernel(points1, kernel, W1):
    feats2, pts_sub, cnt = _kpconv_features(points1, kernel, W1)
    pT_pad = jnp.pad(points1.T, ((0, 0), (0, NPAD - N)))
    xs, ys, zs = pT_pad[0], pT_pad[1], pT_pad[2]
    f_pad = jnp.pad(feats2, ((0, NPAD - N), (0, FD - OUT_FEATS)))
    cntp = jnp.pad(cnt, ((0, VPAD - VOXELS), (0, 0)))
    cnt_bcast = jnp.broadcast_to(cntp, (VPAD, 16))
    feats_sub = _feat_pool_sc(xs, ys, zs, f_pad, cnt_bcast)
    return pts_sub, feats_sub
